# baseline (speedup 1.0000x reference)
import jax
import jax.numpy as jnp
from jax import lax
from jax.experimental import pallas as pl
from jax.experimental.pallas import tpu as pltpu, tpu_sc as plsc

N = 10000
E = 320000
D = 128
NC = 2
NS = 16
CH = 80
BLK = 50
EPT = E // NS
NCHUNK = EPT // CH
NBLK = NCHUNK // BLK
ZR = 1000


def _tc1_body(x_ref, w0_ref, w1_ref, as0_ref, ad0_ref, as1_ref, ad1_ref,
              h0_ref, h1_ref, scal_ref):
    x = x_ref[...]
    h0 = jnp.dot(x, w0_ref[...], preferred_element_type=jnp.float32)
    h1 = jnp.dot(x, w1_ref[...], preferred_element_type=jnp.float32)
    h0_ref[...] = h0
    h1_ref[...] = h1
    scal_ref[0, :] = jnp.sum(h0 * as0_ref[...], axis=-1)
    scal_ref[1, :] = jnp.sum(h0 * ad0_ref[...], axis=-1)
    scal_ref[2, :] = jnp.sum(h1 * as1_ref[...], axis=-1)
    scal_ref[3, :] = jnp.sum(h1 * ad1_ref[...], axis=-1)
    scal_ref[4, :] = jnp.zeros_like(scal_ref[4, :])
    scal_ref[5, :] = jnp.zeros_like(scal_ref[5, :])
    scal_ref[6, :] = jnp.zeros_like(scal_ref[6, :])
    scal_ref[7, :] = jnp.zeros_like(scal_ref[7, :])


def _tc1(x, W0, W1, a_src0, a_dst0, a_src1, a_dst1):
    return pl.pallas_call(
        _tc1_body,
        out_shape=[
            jax.ShapeDtypeStruct((N, D), jnp.float32),
            jax.ShapeDtypeStruct((N, D), jnp.float32),
            jax.ShapeDtypeStruct((8, N), jnp.float32),
        ],
    )(x, W0, W1, a_src0, a_dst0, a_src1, a_dst1)


def _sc_body(h0, h1, as0, ad0, as1, ad1, src_h, dst_h,
             out0, out1, dn0, dn1, ss0, ss1, cnt,
             u_sh, dn_sh, ss_sh, cnt_sh,
             srcb, dstb, rows_a, rows_b, aga, bga, agb, bgb,
             atta, attb, exb, ones_v, zbuf, zbufv,
             sg0, sg1, sr0, sr1, sem):
    c = lax.axis_index("c")
    s = lax.axis_index("s")

    def pick(f):
        def run(tbl0, tbl1, *a):
            @pl.when(c == 0)
            def _():
                f(tbl0, *a)

            @pl.when(c == 1)
            def _():
                f(tbl1, *a)
        return run

    zero16 = jnp.zeros((16,), jnp.float32)

    def zfill(r, carry):
        for f in range(D // 16):
            zbuf[r, pl.ds(16 * f, 16)] = zero16
        return carry

    lax.fori_loop(0, 40, zfill, 0)

    def zfillv(i, carry):
        zbufv[pl.ds(i * 16, 16)] = zero16
        return carry

    lax.fori_loop(0, ZR // 16, zfillv, 0)
    zbufv[pl.ds(ZR - 16, 16)] = zero16

    @pl.when(s < 10)
    def _():
        r0 = s * ZR
        for i in range(ZR // 40):
            pltpu.sync_copy(zbuf, u_sh.at[pl.ds(r0 + 40 * i, 40)])
        pltpu.sync_copy(zbufv, dn_sh.at[pl.ds(r0, ZR)])
        pltpu.sync_copy(zbufv, ss_sh.at[pl.ds(r0, ZR)])
        pltpu.sync_copy(zbufv, cnt_sh.at[pl.ds(r0, ZR)])

    for i in range(CH // 16):
        ones_v[pl.ds(16 * i, 16)] = jnp.full((16,), 1.0, jnp.float32)

    plsc.subcore_barrier()


    def start_g(x, ag, bg, rows, sg):
        def go(tas, tad, th):
            pltpu.async_copy(tas.at[srcb.at[x]], ag, sg)
            pltpu.async_copy(tad.at[dstb.at[x]], bg, sg)
            pltpu.async_copy(th.at[srcb.at[x]], rows, sg)

        @pl.when(c == 0)
        def _():
            go(as0, ad0, h0)

        @pl.when(c == 1)
        def _():
            go(as1, ad1, h1)

    def wait_g(x, ag, bg, rows, sg):
        def wg(tas, tad, th):
            pltpu.make_async_copy(tas.at[srcb.at[x]], ag, sg).wait()
            pltpu.make_async_copy(tad.at[dstb.at[x]], bg, sg).wait()
            pltpu.make_async_copy(th.at[srcb.at[x]], rows, sg).wait()

        @pl.when(c == 0)
        def _():
            wg(as0, ad0, h0)

        @pl.when(c == 1)
        def _():
            wg(as1, ad1, h1)

    def compute_ex(ag, bg, att):
        for i in range(CH // 16):
            sl = pl.ds(16 * i, 16)
            a = ag[sl] + bg[sl]
            a = jnp.maximum(a, 0.2 * a)
            att[sl] = jnp.exp(a)

    def mul_rows(rows, att):
        def mbody(g, carry2):
            ev16 = att[pl.ds(16 * g, 16)]
            for lane in range(16):
                e = ev16[lane]
                r = 16 * g + lane
                for f in range(D // 16):
                    sl = pl.ds(16 * f, 16)
                    rows[r, sl] = rows[r, sl] * e
            return carry2

        lax.fori_loop(0, CH // 16, mbody, 0)

    def start_sc(x, rows, ssem):
        pltpu.async_copy(rows, u_sh.at[dstb.at[x]], ssem, add=True)

    def wait_sc(x, rows, ssem):
        pltpu.make_async_copy(rows, u_sh.at[dstb.at[x]], ssem).wait()

    def post_sc(x, att):
        pltpu.sync_copy(att, dn_sh.at[dstb.at[x]], add=True)

    def block(b, carry):
        pltpu.sync_copy(src_h.at[s * NBLK + b], srcb)
        pltpu.sync_copy(dst_h.at[s * NBLK + b], dstb)
        start_g(0, aga, bga, rows_a, sg0)

        def pair(k, carry2):
            a = 2 * k
            bl = 2 * k + 1
            wait_g(a, aga, bga, rows_a, sg0)
            compute_ex(aga, bga, atta)

            @pl.when(k > 0)
            def _():
                wait_sc(bl - 2, rows_b, sr1)
                post_sc(bl - 2, attb)

            start_g(bl, agb, bgb, rows_b, sg1)
            mul_rows(rows_a, atta)
            start_sc(a, rows_a, sr0)
            wait_g(bl, agb, bgb, rows_b, sg1)
            compute_ex(agb, bgb, attb)
            mul_rows(rows_b, attb)
            wait_sc(a, rows_a, sr0)
            post_sc(a, atta)

            @pl.when(k < BLK // 2 - 1)
            def _():
                start_g(a + 2, aga, bga, rows_a, sg0)

            start_sc(bl, rows_b, sr1)
            return carry2

        lax.fori_loop(0, BLK // 2, pair, 0)
        wait_sc(BLK - 1, rows_b, sr1)
        post_sc(BLK - 1, attb)
        return carry

    lax.fori_loop(0, NBLK, block, 0)
    plsc.subcore_barrier()

    def start_g2(x, ag, bg, sg):
        def go2(tas, tad):
            pltpu.async_copy(tas.at[srcb.at[x]], ag, sg)
            pltpu.async_copy(tad.at[dstb.at[x]], bg, sg)

        @pl.when(c == 0)
        def _():
            go2(as0, ad0)

        @pl.when(c == 1)
        def _():
            go2(as1, ad1)

    def wait_g2(x, ag, bg, sg):
        def wg2(tas, tad):
            pltpu.make_async_copy(tas.at[srcb.at[x]], ag, sg).wait()
            pltpu.make_async_copy(tad.at[dstb.at[x]], bg, sg).wait()

        @pl.when(c == 0)
        def _():
            wg2(as0, ad0)

        @pl.when(c == 1)
        def _():
            wg2(as1, ad1)

    def block2(b, carry):
        pltpu.sync_copy(src_h.at[s * NBLK + b], srcb)
        pltpu.sync_copy(dst_h.at[s * NBLK + b], dstb)
        start_g2(0, aga, bga, sg0)

        def p2chunk(x, ag, bg, sg):
            wait_g2(x, ag, bg, sg)
            compute_ex(ag, bg, exb)
            pltpu.async_copy(dn_sh.at[dstb.at[x]], atta, sem).wait()
            for i in range(CH // 16):
                sl = pl.ds(16 * i, 16)
                exb[sl] = exb[sl] / (atta[sl] + 1e-16)
            pltpu.sync_copy(exb, ss_sh.at[srcb.at[x]], add=True)

            @pl.when(c == 0)
            def _():
                pltpu.sync_copy(ones_v, cnt_sh.at[srcb.at[x]], add=True)

        def pair2(k, carry2):
            a = 2 * k
            bl = 2 * k + 1
            start_g2(bl, agb, bgb, sg1)
            p2chunk(a, aga, bga, sg0)

            @pl.when(k < BLK // 2 - 1)
            def _():
                start_g2(a + 2, aga, bga, sg0)

            p2chunk(bl, agb, bgb, sg1)
            return carry2

        lax.fori_loop(0, BLK // 2, pair2, 0)
        return carry

    lax.fori_loop(0, NBLK, block2, 0)
    plsc.subcore_barrier()

    @pl.when(s < 10)
    def _():
        r0 = s * ZR

        def d_out(o):
            def dchunk(i, carry):
                sl = pl.ds(r0 + 40 * i, 40)
                pltpu.sync_copy(u_sh.at[sl], zbuf)
                pltpu.sync_copy(zbuf, o.at[sl])
                return carry

            lax.fori_loop(0, ZR // 40, dchunk, 0)

        def d_vec(sh, o):
            sl = pl.ds(r0, ZR)
            pltpu.sync_copy(sh.at[sl], zbufv)
            pltpu.sync_copy(zbufv, o.at[sl])

        pick(d_out)(out0, out1)
        pick(lambda o: d_vec(dn_sh, o))(dn0, dn1)
        pick(lambda o: d_vec(ss_sh, o))(ss0, ss1)

        @pl.when(c == 0)
        def _():
            d_vec(cnt_sh, cnt)


def _sc_call(h0, h1, as0, ad0, as1, ad1, src_r, dst_r):
    mesh = plsc.VectorSubcoreMesh(core_axis_name="c", subcore_axis_name="s",
                                  num_cores=NC, num_subcores=NS)
    f32 = jnp.float32
    out_type = [
        jax.ShapeDtypeStruct((N, D), f32),
        jax.ShapeDtypeStruct((N, D), f32),
        jax.ShapeDtypeStruct((N,), f32),
        jax.ShapeDtypeStruct((N,), f32),
        jax.ShapeDtypeStruct((N,), f32),
        jax.ShapeDtypeStruct((N,), f32),
        jax.ShapeDtypeStruct((N,), f32),
    ]
    scratch = [
        pltpu.VMEM_SHARED((N, D), f32),
        pltpu.VMEM_SHARED((N,), f32),
        pltpu.VMEM_SHARED((N,), f32),
        pltpu.VMEM_SHARED((N,), f32),
        pltpu.VMEM((BLK, CH), jnp.int32),
        pltpu.VMEM((BLK, CH), jnp.int32),
        pltpu.VMEM((CH, D), f32),
        pltpu.VMEM((CH, D), f32),
        pltpu.VMEM((CH,), f32),
        pltpu.VMEM((CH,), f32),
        pltpu.VMEM((CH,), f32),
        pltpu.VMEM((CH,), f32),
        pltpu.VMEM((CH,), f32),
        pltpu.VMEM((CH,), f32),
        pltpu.VMEM((CH,), f32),
        pltpu.VMEM((CH,), f32),
        pltpu.VMEM((40, D), f32),
        pltpu.VMEM((ZR,), f32),
        pltpu.SemaphoreType.DMA,
        pltpu.SemaphoreType.DMA,
        pltpu.SemaphoreType.DMA,
        pltpu.SemaphoreType.DMA,
        pltpu.SemaphoreType.DMA,
    ]
    fn = pl.kernel(_sc_body, out_type=out_type, mesh=mesh,
                   scratch_types=scratch)
    return fn(h0, h1, as0, ad0, as1, ad1, src_r, dst_r)


def _tc2_body(o0_ref, o1_ref, dn0_ref, dn1_ref, ss0_ref, ss1_ref, cnt_ref,
              b0_ref, b1_ref, feat_ref, scores_ref):
    eps = 1e-16
    o0 = o0_ref[...] / (dn0_ref[...] + eps) + b0_ref[...]
    o1 = o1_ref[...] / (dn1_ref[...] + eps) + b1_ref[...]
    f0 = jnp.maximum(o0, 0.01 * o0)
    f1 = jnp.maximum(o1, 0.01 * o1)
    feat_ref[...] = f0 + f1
    ssum = ss0_ref[...] + ss1_ref[...]
    scores_ref[...] = ssum / jnp.maximum(cnt_ref[...], 1.0)


def _tc2(out0, out1, dn0, dn1, ss0, ss1, cnt, b0, b1):
    f32 = jnp.float32
    return pl.pallas_call(
        _tc2_body,
        out_shape=[
            jax.ShapeDtypeStruct((N, D), f32),
            jax.ShapeDtypeStruct((N, 1), f32),
        ],
    )(out0, out1, dn0, dn1, ss0, ss1, cnt, b0, b1)


def kernel(x, edge_index, W0, a_src0, a_dst0, b0, W1, a_src1, a_dst1, b1):
    src_r = edge_index[0].reshape(NS * NBLK, BLK, CH)
    dst_r = edge_index[1].reshape(NS * NBLK, BLK, CH)
    h0, h1, scal = _tc1(x, W0, W1,
                        a_src0.reshape(1, D), a_dst0.reshape(1, D),
                        a_src1.reshape(1, D), a_dst1.reshape(1, D))
    as0, ad0, as1, ad1 = scal[0], scal[1], scal[2], scal[3]
    out0, out1, dn0, dn1, ss0, ss1, cnt = _sc_call(
        h0, h1, as0, ad0, as1, ad1, src_r, dst_r)
    feat, scores = _tc2(out0, out1,
                        dn0.reshape(N, 1), dn1.reshape(N, 1),
                        ss0.reshape(N, 1), ss1.reshape(N, 1),
                        cnt.reshape(N, 1),
                        b0.reshape(1, D), b1.reshape(1, D))
    return (feat, scores.reshape(N))


# speedup vs baseline: 1.1008x; 1.1008x over previous
import jax
import jax.numpy as jnp
from jax import lax
from jax.experimental import pallas as pl
from jax.experimental.pallas import tpu as pltpu, tpu_sc as plsc

N = 10000
E = 320000
D = 128
NC = 2
NS = 16
CH = 80
BLK = 50
EPT = E // NS
NCHUNK = EPT // CH
NBLK = NCHUNK // BLK
ZR = 1000


def _tc1_body(x_ref, w0_ref, w1_ref, as0_ref, ad0_ref, as1_ref, ad1_ref,
              h0_ref, h1_ref, scal_ref):
    x = x_ref[...]
    h0 = jnp.dot(x, w0_ref[...], preferred_element_type=jnp.float32)
    h1 = jnp.dot(x, w1_ref[...], preferred_element_type=jnp.float32)
    h0_ref[...] = h0
    h1_ref[...] = h1
    scal_ref[0, :] = jnp.sum(h0 * as0_ref[...], axis=-1)
    scal_ref[1, :] = jnp.sum(h0 * ad0_ref[...], axis=-1)
    scal_ref[2, :] = jnp.sum(h1 * as1_ref[...], axis=-1)
    scal_ref[3, :] = jnp.sum(h1 * ad1_ref[...], axis=-1)
    scal_ref[4, :] = jnp.zeros_like(scal_ref[4, :])
    scal_ref[5, :] = jnp.zeros_like(scal_ref[5, :])
    scal_ref[6, :] = jnp.zeros_like(scal_ref[6, :])
    scal_ref[7, :] = jnp.zeros_like(scal_ref[7, :])


def _tc1(x, W0, W1, a_src0, a_dst0, a_src1, a_dst1):
    return pl.pallas_call(
        _tc1_body,
        out_shape=[
            jax.ShapeDtypeStruct((N, D), jnp.float32),
            jax.ShapeDtypeStruct((N, D), jnp.float32),
            jax.ShapeDtypeStruct((8, N), jnp.float32),
        ],
    )(x, W0, W1, a_src0, a_dst0, a_src1, a_dst1)


def _sc_body(h0, h1, as0, ad0, as1, ad1, src_h, dst_h,
             out0, out1, dn0, dn1, ss0, ss1, cnt,
             u_sh, dn_sh, ss_sh, cnt_sh,
             srcb, dstb, rows_a, rows_b, aga, bga, agb, bgb,
             atta, attb, exb, ones_v, zbuf, zbufv,
             sg0, sg1, sr0, sr1, sem):
    c = lax.axis_index("c")
    s = lax.axis_index("s")

    def pick(f):
        def run(tbl0, tbl1, *a):
            @pl.when(c == 0)
            def _():
                f(tbl0, *a)

            @pl.when(c == 1)
            def _():
                f(tbl1, *a)
        return run

    zero16 = jnp.zeros((16,), jnp.float32)

    def zfill(r, carry):
        for f in range(D // 16):
            zbuf[r, pl.ds(16 * f, 16)] = zero16
        return carry

    lax.fori_loop(0, 40, zfill, 0)

    def zfillv(i, carry):
        zbufv[pl.ds(i * 16, 16)] = zero16
        return carry

    lax.fori_loop(0, ZR // 16, zfillv, 0)
    zbufv[pl.ds(ZR - 16, 16)] = zero16

    @pl.when(s < 10)
    def _():
        r0 = s * ZR
        for i in range(ZR // 40):
            pltpu.sync_copy(zbuf, u_sh.at[pl.ds(r0 + 40 * i, 40)])
        pltpu.sync_copy(zbufv, dn_sh.at[pl.ds(r0, ZR)])
        pltpu.sync_copy(zbufv, ss_sh.at[pl.ds(r0, ZR)])
        pltpu.sync_copy(zbufv, cnt_sh.at[pl.ds(r0, ZR)])

    for i in range(CH // 16):
        ones_v[pl.ds(16 * i, 16)] = jnp.full((16,), 1.0, jnp.float32)

    plsc.subcore_barrier()


    def start_g(x, ag, bg, rows, sg):
        def go(tas, tad, th):
            pltpu.async_copy(tas.at[srcb.at[x]], ag, sg)
            pltpu.async_copy(tad.at[dstb.at[x]], bg, sg)
            pltpu.async_copy(th.at[srcb.at[x]], rows, sg)

        @pl.when(c == 0)
        def _():
            go(as0, ad0, h0)

        @pl.when(c == 1)
        def _():
            go(as1, ad1, h1)

    def wait_g(x, ag, bg, rows, sg):
        def wg(tas, tad, th):
            pltpu.make_async_copy(tas.at[srcb.at[x]], ag, sg).wait()
            pltpu.make_async_copy(tad.at[dstb.at[x]], bg, sg).wait()
            pltpu.make_async_copy(th.at[srcb.at[x]], rows, sg).wait()

        @pl.when(c == 0)
        def _():
            wg(as0, ad0, h0)

        @pl.when(c == 1)
        def _():
            wg(as1, ad1, h1)

    def compute_ex(ag, bg, att):
        for i in range(CH // 16):
            sl = pl.ds(16 * i, 16)
            a = ag[sl] + bg[sl]
            a = jnp.maximum(a, 0.2 * a)
            att[sl] = jnp.exp(a)

    def mul_rows(rows, att):
        def mbody(g, carry2):
            ev16 = att[pl.ds(16 * g, 16)]
            for lane in range(16):
                e = ev16[lane]
                r = 16 * g + lane
                for f in range(D // 16):
                    sl = pl.ds(16 * f, 16)
                    rows[r, sl] = rows[r, sl] * e
            return carry2

        lax.fori_loop(0, CH // 16, mbody, 0)

    def scatter_sc(x, rows, att):
        pltpu.sync_copy(rows, u_sh.at[dstb.at[x]], add=True)
        pltpu.sync_copy(att, dn_sh.at[dstb.at[x]], add=True)

    def block(b, carry):
        pltpu.sync_copy(src_h.at[s * NBLK + b], srcb)
        pltpu.sync_copy(dst_h.at[s * NBLK + b], dstb)
        start_g(0, aga, bga, rows_a, sg0)

        def pair(k, carry2):
            a = 2 * k
            bl = 2 * k + 1
            wait_g(a, aga, bga, rows_a, sg0)
            compute_ex(aga, bga, atta)
            start_g(bl, agb, bgb, rows_b, sg1)
            mul_rows(rows_a, atta)
            scatter_sc(a, rows_a, atta)
            wait_g(bl, agb, bgb, rows_b, sg1)
            compute_ex(agb, bgb, attb)

            @pl.when(k < BLK // 2 - 1)
            def _():
                start_g(a + 2, aga, bga, rows_a, sg0)

            mul_rows(rows_b, attb)
            scatter_sc(bl, rows_b, attb)
            return carry2

        lax.fori_loop(0, BLK // 2, pair, 0)
        return carry

    lax.fori_loop(0, NBLK, block, 0)
    plsc.subcore_barrier()

    def start_g2(x, ag, bg, sg):
        def go2(tas, tad):
            pltpu.async_copy(tas.at[srcb.at[x]], ag, sg)
            pltpu.async_copy(tad.at[dstb.at[x]], bg, sg)

        @pl.when(c == 0)
        def _():
            go2(as0, ad0)

        @pl.when(c == 1)
        def _():
            go2(as1, ad1)

    def wait_g2(x, ag, bg, sg):
        def wg2(tas, tad):
            pltpu.make_async_copy(tas.at[srcb.at[x]], ag, sg).wait()
            pltpu.make_async_copy(tad.at[dstb.at[x]], bg, sg).wait()

        @pl.when(c == 0)
        def _():
            wg2(as0, ad0)

        @pl.when(c == 1)
        def _():
            wg2(as1, ad1)

    def block2(b, carry):
        pltpu.sync_copy(src_h.at[s * NBLK + b], srcb)
        pltpu.sync_copy(dst_h.at[s * NBLK + b], dstb)
        start_g2(0, aga, bga, sg0)

        def p2chunk(x, ag, bg, sg):
            wait_g2(x, ag, bg, sg)
            compute_ex(ag, bg, exb)
            pltpu.async_copy(dn_sh.at[dstb.at[x]], atta, sem).wait()
            for i in range(CH // 16):
                sl = pl.ds(16 * i, 16)
                exb[sl] = exb[sl] / (atta[sl] + 1e-16)
            pltpu.sync_copy(exb, ss_sh.at[srcb.at[x]], add=True)

            @pl.when(c == 0)
            def _():
                pltpu.sync_copy(ones_v, cnt_sh.at[srcb.at[x]], add=True)

        def pair2(k, carry2):
            a = 2 * k
            bl = 2 * k + 1
            start_g2(bl, agb, bgb, sg1)
            p2chunk(a, aga, bga, sg0)

            @pl.when(k < BLK // 2 - 1)
            def _():
                start_g2(a + 2, aga, bga, sg0)

            p2chunk(bl, agb, bgb, sg1)
            return carry2

        lax.fori_loop(0, BLK // 2, pair2, 0)
        return carry

    lax.fori_loop(0, NBLK, block2, 0)
    plsc.subcore_barrier()

    @pl.when(s < 10)
    def _():
        r0 = s * ZR

        def d_out(o):
            def dchunk(i, carry):
                sl = pl.ds(r0 + 40 * i, 40)
                pltpu.sync_copy(u_sh.at[sl], zbuf)
                pltpu.sync_copy(zbuf, o.at[sl])
                return carry

            lax.fori_loop(0, ZR // 40, dchunk, 0)

        def d_vec(sh, o):
            sl = pl.ds(r0, ZR)
            pltpu.sync_copy(sh.at[sl], zbufv)
            pltpu.sync_copy(zbufv, o.at[sl])

        pick(d_out)(out0, out1)
        pick(lambda o: d_vec(dn_sh, o))(dn0, dn1)
        pick(lambda o: d_vec(ss_sh, o))(ss0, ss1)

        @pl.when(c == 0)
        def _():
            d_vec(cnt_sh, cnt)


def _sc_call(h0, h1, as0, ad0, as1, ad1, src_r, dst_r):
    mesh = plsc.VectorSubcoreMesh(core_axis_name="c", subcore_axis_name="s",
                                  num_cores=NC, num_subcores=NS)
    f32 = jnp.float32
    out_type = [
        jax.ShapeDtypeStruct((N, D), f32),
        jax.ShapeDtypeStruct((N, D), f32),
        jax.ShapeDtypeStruct((N,), f32),
        jax.ShapeDtypeStruct((N,), f32),
        jax.ShapeDtypeStruct((N,), f32),
        jax.ShapeDtypeStruct((N,), f32),
        jax.ShapeDtypeStruct((N,), f32),
    ]
    scratch = [
        pltpu.VMEM_SHARED((N, D), f32),
        pltpu.VMEM_SHARED((N,), f32),
        pltpu.VMEM_SHARED((N,), f32),
        pltpu.VMEM_SHARED((N,), f32),
        pltpu.VMEM((BLK, CH), jnp.int32),
        pltpu.VMEM((BLK, CH), jnp.int32),
        pltpu.VMEM((CH, D), f32),
        pltpu.VMEM((CH, D), f32),
        pltpu.VMEM((CH,), f32),
        pltpu.VMEM((CH,), f32),
        pltpu.VMEM((CH,), f32),
        pltpu.VMEM((CH,), f32),
        pltpu.VMEM((CH,), f32),
        pltpu.VMEM((CH,), f32),
        pltpu.VMEM((CH,), f32),
        pltpu.VMEM((CH,), f32),
        pltpu.VMEM((40, D), f32),
        pltpu.VMEM((ZR,), f32),
        pltpu.SemaphoreType.DMA,
        pltpu.SemaphoreType.DMA,
        pltpu.SemaphoreType.DMA,
        pltpu.SemaphoreType.DMA,
        pltpu.SemaphoreType.DMA,
    ]
    fn = pl.kernel(_sc_body, out_type=out_type, mesh=mesh,
                   scratch_types=scratch)
    return fn(h0, h1, as0, ad0, as1, ad1, src_r, dst_r)


def _tc2_body(o0_ref, o1_ref, dn0_ref, dn1_ref, ss0_ref, ss1_ref, cnt_ref,
              b0_ref, b1_ref, feat_ref, scores_ref):
    eps = 1e-16
    o0 = o0_ref[...] / (dn0_ref[...] + eps) + b0_ref[...]
    o1 = o1_ref[...] / (dn1_ref[...] + eps) + b1_ref[...]
    f0 = jnp.maximum(o0, 0.01 * o0)
    f1 = jnp.maximum(o1, 0.01 * o1)
    feat_ref[...] = f0 + f1
    ssum = ss0_ref[...] + ss1_ref[...]
    scores_ref[...] = ssum / jnp.maximum(cnt_ref[...], 1.0)


def _tc2(out0, out1, dn0, dn1, ss0, ss1, cnt, b0, b1):
    f32 = jnp.float32
    return pl.pallas_call(
        _tc2_body,
        out_shape=[
            jax.ShapeDtypeStruct((N, D), f32),
            jax.ShapeDtypeStruct((N, 1), f32),
        ],
    )(out0, out1, dn0, dn1, ss0, ss1, cnt, b0, b1)


def kernel(x, edge_index, W0, a_src0, a_dst0, b0, W1, a_src1, a_dst1, b1):
    src_r = edge_index[0].reshape(NS * NBLK, BLK, CH)
    dst_r = edge_index[1].reshape(NS * NBLK, BLK, CH)
    h0, h1, scal = _tc1(x, W0, W1,
                        a_src0.reshape(1, D), a_dst0.reshape(1, D),
                        a_src1.reshape(1, D), a_dst1.reshape(1, D))
    as0, ad0, as1, ad1 = scal[0], scal[1], scal[2], scal[3]
    out0, out1, dn0, dn1, ss0, ss1, cnt = _sc_call(
        h0, h1, as0, ad0, as1, ad1, src_r, dst_r)
    feat, scores = _tc2(out0, out1,
                        dn0.reshape(N, 1), dn1.reshape(N, 1),
                        ss0.reshape(N, 1), ss1.reshape(N, 1),
                        cnt.reshape(N, 1),
                        b0.reshape(1, D), b1.reshape(1, D))
    return (feat, scores.reshape(N))


# speedup vs baseline: 1.2192x; 1.1075x over previous
import jax
import jax.numpy as jnp
from jax import lax
from jax.experimental import pallas as pl
from jax.experimental.pallas import tpu as pltpu, tpu_sc as plsc

N = 10000
E = 320000
D = 128
NC = 2
NS = 16
CH = 80
BLK = 50
EPT = E // NS
NCHUNK = EPT // CH
NBLK = NCHUNK // BLK
ZR = 1000


def _tc1_body(x_ref, w0_ref, w1_ref, as0_ref, ad0_ref, as1_ref, ad1_ref,
              h0_ref, h1_ref, scal_ref):
    x = x_ref[...]
    h0 = jnp.dot(x, w0_ref[...], preferred_element_type=jnp.float32)
    h1 = jnp.dot(x, w1_ref[...], preferred_element_type=jnp.float32)
    h0_ref[...] = h0
    h1_ref[...] = h1
    scal_ref[0, :] = jnp.sum(h0 * as0_ref[...], axis=-1)
    scal_ref[1, :] = jnp.sum(h0 * ad0_ref[...], axis=-1)
    scal_ref[2, :] = jnp.sum(h1 * as1_ref[...], axis=-1)
    scal_ref[3, :] = jnp.sum(h1 * ad1_ref[...], axis=-1)
    scal_ref[4, :] = jnp.zeros_like(scal_ref[4, :])
    scal_ref[5, :] = jnp.zeros_like(scal_ref[5, :])
    scal_ref[6, :] = jnp.zeros_like(scal_ref[6, :])
    scal_ref[7, :] = jnp.zeros_like(scal_ref[7, :])


def _tc1(x, W0, W1, a_src0, a_dst0, a_src1, a_dst1):
    return pl.pallas_call(
        _tc1_body,
        out_shape=[
            jax.ShapeDtypeStruct((N, D), jnp.float32),
            jax.ShapeDtypeStruct((N, D), jnp.float32),
            jax.ShapeDtypeStruct((8, N), jnp.float32),
        ],
    )(x, W0, W1, a_src0, a_dst0, a_src1, a_dst1)


def _sc_body(h0, h1, as0, ad0, as1, ad1, src_h, dst_h,
             out0, out1, dn0, dn1, ss0, ss1, cnt0, cnt1, exh,
             u_sh, dn_sh, ss_sh, cnt_sh,
             srcb, dstb, rows_a, rows_b, aga, bga, agb, bgb,
             atta, attb, exb, ones_v, zbuf, zbufv,
             sg0, sg1, sr0, sr1, sem):
    c = lax.axis_index("c")
    s = lax.axis_index("s")

    def pick(f):
        def run(tbl0, tbl1, *a):
            @pl.when(c == 0)
            def _():
                f(tbl0, *a)

            @pl.when(c == 1)
            def _():
                f(tbl1, *a)
        return run

    zero16 = jnp.zeros((16,), jnp.float32)

    def zfill(r, carry):
        for f in range(D // 16):
            zbuf[r, pl.ds(16 * f, 16)] = zero16
        return carry

    lax.fori_loop(0, 40, zfill, 0)

    def zfillv(i, carry):
        zbufv[pl.ds(i * 16, 16)] = zero16
        return carry

    lax.fori_loop(0, ZR // 16, zfillv, 0)
    zbufv[pl.ds(ZR - 16, 16)] = zero16

    @pl.when(s < 10)
    def _():
        r0 = s * ZR
        for i in range(ZR // 40):
            pltpu.sync_copy(zbuf, u_sh.at[pl.ds(r0 + 40 * i, 40)])
        pltpu.sync_copy(zbufv, dn_sh.at[pl.ds(r0, ZR)])
        pltpu.sync_copy(zbufv, ss_sh.at[pl.ds(r0, ZR)])
        pltpu.sync_copy(zbufv, cnt_sh.at[pl.ds(r0, ZR)])

    for i in range(CH // 16):
        ones_v[pl.ds(16 * i, 16)] = jnp.full((16,), 1.0, jnp.float32)

    plsc.subcore_barrier()

    exbase = c * E + s * EPT

    def start_g(x, ag, bg, rows, sg):
        def go(tas, tad, th):
            pltpu.async_copy(tas.at[srcb.at[x]], ag, sg)
            pltpu.async_copy(tad.at[dstb.at[x]], bg, sg)
            pltpu.async_copy(th.at[srcb.at[x]], rows, sg)

        @pl.when(c == 0)
        def _():
            go(as0, ad0, h0)

        @pl.when(c == 1)
        def _():
            go(as1, ad1, h1)

    def wait_g(x, ag, bg, rows, sg):
        def wg(tas, tad, th):
            pltpu.make_async_copy(tas.at[srcb.at[x]], ag, sg).wait()
            pltpu.make_async_copy(tad.at[dstb.at[x]], bg, sg).wait()
            pltpu.make_async_copy(th.at[srcb.at[x]], rows, sg).wait()

        @pl.when(c == 0)
        def _():
            wg(as0, ad0, h0)

        @pl.when(c == 1)
        def _():
            wg(as1, ad1, h1)

    def compute_ex(ag, bg, att):
        for i in range(CH // 16):
            sl = pl.ds(16 * i, 16)
            a = ag[sl] + bg[sl]
            a = jnp.maximum(a, 0.2 * a)
            att[sl] = jnp.exp(a)

    def mul_rows(rows, att):
        def mbody(g, carry2):
            ev16 = att[pl.ds(16 * g, 16)]
            for lane in range(16):
                e = ev16[lane]
                r = 16 * g + lane
                for f in range(D // 16):
                    sl = pl.ds(16 * f, 16)
                    rows[r, sl] = rows[r, sl] * e
            return carry2

        lax.fori_loop(0, CH // 16, mbody, 0)

    def scatter_sc(bgl, x, rows, att, cnt_core):
        pltpu.sync_copy(rows, u_sh.at[dstb.at[x]], add=True)
        pltpu.sync_copy(att, dn_sh.at[dstb.at[x]], add=True)
        pltpu.sync_copy(att, exh.at[pl.ds(exbase + bgl * CH, CH)])

        @pl.when(c == cnt_core)
        def _():
            pltpu.sync_copy(ones_v, cnt_sh.at[srcb.at[x]], add=True)

    def block(b, carry):
        pltpu.sync_copy(src_h.at[s * NBLK + b], srcb)
        pltpu.sync_copy(dst_h.at[s * NBLK + b], dstb)
        base = b * BLK
        start_g(0, aga, bga, rows_a, sg0)

        def pair(k, carry2):
            a = 2 * k
            bl = 2 * k + 1
            wait_g(a, aga, bga, rows_a, sg0)
            compute_ex(aga, bga, atta)
            start_g(bl, agb, bgb, rows_b, sg1)
            mul_rows(rows_a, atta)
            scatter_sc(base + a, a, rows_a, atta, 0)
            wait_g(bl, agb, bgb, rows_b, sg1)
            compute_ex(agb, bgb, attb)

            @pl.when(k < BLK // 2 - 1)
            def _():
                start_g(a + 2, aga, bga, rows_a, sg0)

            mul_rows(rows_b, attb)
            scatter_sc(base + bl, bl, rows_b, attb, 1)
            return carry2

        lax.fori_loop(0, BLK // 2, pair, 0)
        return carry

    lax.fori_loop(0, NBLK, block, 0)
    plsc.subcore_barrier()

    def p2start(bgl, x, ebuf, dbuf, sg):
        pltpu.async_copy(exh.at[pl.ds(exbase + bgl * CH, CH)], ebuf, sg)

    def p2wait(bgl, x, ebuf, dbuf, sg):
        pltpu.make_async_copy(
            exh.at[pl.ds(exbase + bgl * CH, CH)], ebuf, sg).wait()

    def p2chunk(bgl, x, ebuf, dbuf, sg):
        p2wait(bgl, x, ebuf, dbuf, sg)
        pltpu.async_copy(dn_sh.at[dstb.at[x]], dbuf, sem).wait()
        for i in range(CH // 16):
            sl = pl.ds(16 * i, 16)
            atta[sl] = ebuf[sl] / (dbuf[sl] + 1e-16)
        pltpu.sync_copy(atta, ss_sh.at[srcb.at[x]], add=True)

    def block2(b, carry):
        pltpu.sync_copy(src_h.at[s * NBLK + b], srcb)
        pltpu.sync_copy(dst_h.at[s * NBLK + b], dstb)
        base = b * BLK
        p2start(base, 0, exb, bga, sg0)

        def pair2(k, carry2):
            a = 2 * k
            bl = 2 * k + 1
            p2start(base + bl, bl, agb, bgb, sg1)
            p2chunk(base + a, a, exb, bga, sg0)

            @pl.when(k < BLK // 2 - 1)
            def _():
                p2start(base + a + 2, a + 2, exb, bga, sg0)

            p2chunk(base + bl, bl, agb, bgb, sg1)
            return carry2

        lax.fori_loop(0, BLK // 2, pair2, 0)
        return carry

    lax.fori_loop(0, NBLK, block2, 0)
    plsc.subcore_barrier()

    @pl.when(s < 10)
    def _():
        r0 = s * ZR

        def d_out(o):
            def dchunk(i, carry):
                sl = pl.ds(r0 + 40 * i, 40)
                pltpu.sync_copy(u_sh.at[sl], zbuf)
                pltpu.sync_copy(zbuf, o.at[sl])
                return carry

            lax.fori_loop(0, ZR // 40, dchunk, 0)

        def d_vec(sh, o):
            sl = pl.ds(r0, ZR)
            pltpu.sync_copy(sh.at[sl], zbufv)
            pltpu.sync_copy(zbufv, o.at[sl])

        pick(d_out)(out0, out1)
        pick(lambda o: d_vec(dn_sh, o))(dn0, dn1)
        pick(lambda o: d_vec(ss_sh, o))(ss0, ss1)
        pick(lambda o: d_vec(cnt_sh, o))(cnt0, cnt1)


def _sc_call(h0, h1, as0, ad0, as1, ad1, src_r, dst_r):
    mesh = plsc.VectorSubcoreMesh(core_axis_name="c", subcore_axis_name="s",
                                  num_cores=NC, num_subcores=NS)
    f32 = jnp.float32
    out_type = [
        jax.ShapeDtypeStruct((N, D), f32),
        jax.ShapeDtypeStruct((N, D), f32),
        jax.ShapeDtypeStruct((N,), f32),
        jax.ShapeDtypeStruct((N,), f32),
        jax.ShapeDtypeStruct((N,), f32),
        jax.ShapeDtypeStruct((N,), f32),
        jax.ShapeDtypeStruct((N,), f32),
        jax.ShapeDtypeStruct((N,), f32),
        jax.ShapeDtypeStruct((2 * E,), f32),
    ]
    scratch = [
        pltpu.VMEM_SHARED((N, D), f32),
        pltpu.VMEM_SHARED((N,), f32),
        pltpu.VMEM_SHARED((N,), f32),
        pltpu.VMEM_SHARED((N,), f32),
        pltpu.VMEM((BLK, CH), jnp.int32),
        pltpu.VMEM((BLK, CH), jnp.int32),
        pltpu.VMEM((CH, D), f32),
        pltpu.VMEM((CH, D), f32),
        pltpu.VMEM((CH,), f32),
        pltpu.VMEM((CH,), f32),
        pltpu.VMEM((CH,), f32),
        pltpu.VMEM((CH,), f32),
        pltpu.VMEM((CH,), f32),
        pltpu.VMEM((CH,), f32),
        pltpu.VMEM((CH,), f32),
        pltpu.VMEM((CH,), f32),
        pltpu.VMEM((40, D), f32),
        pltpu.VMEM((ZR,), f32),
        pltpu.SemaphoreType.DMA,
        pltpu.SemaphoreType.DMA,
        pltpu.SemaphoreType.DMA,
        pltpu.SemaphoreType.DMA,
        pltpu.SemaphoreType.DMA,
    ]
    fn = pl.kernel(_sc_body, out_type=out_type, mesh=mesh,
                   scratch_types=scratch)
    return fn(h0, h1, as0, ad0, as1, ad1, src_r, dst_r)


def _tc2_body(o0_ref, o1_ref, dn0_ref, dn1_ref, ss0_ref, ss1_ref,
              cnt0_ref, cnt1_ref, b0_ref, b1_ref, feat_ref, scores_ref):
    eps = 1e-16
    o0 = o0_ref[...] / (dn0_ref[...] + eps) + b0_ref[...]
    o1 = o1_ref[...] / (dn1_ref[...] + eps) + b1_ref[...]
    f0 = jnp.maximum(o0, 0.01 * o0)
    f1 = jnp.maximum(o1, 0.01 * o1)
    feat_ref[...] = f0 + f1
    ssum = ss0_ref[...] + ss1_ref[...]
    cnt = cnt0_ref[...] + cnt1_ref[...]
    scores_ref[...] = ssum / jnp.maximum(cnt, 1.0)


def _tc2(out0, out1, dn0, dn1, ss0, ss1, cnt0, cnt1, b0, b1):
    f32 = jnp.float32
    return pl.pallas_call(
        _tc2_body,
        out_shape=[
            jax.ShapeDtypeStruct((N, D), f32),
            jax.ShapeDtypeStruct((N, 1), f32),
        ],
    )(out0, out1, dn0, dn1, ss0, ss1, cnt0, cnt1, b0, b1)


def kernel(x, edge_index, W0, a_src0, a_dst0, b0, W1, a_src1, a_dst1, b1):
    src_r = edge_index[0].reshape(NS * NBLK, BLK, CH)
    dst_r = edge_index[1].reshape(NS * NBLK, BLK, CH)
    h0, h1, scal = _tc1(x, W0, W1,
                        a_src0.reshape(1, D), a_dst0.reshape(1, D),
                        a_src1.reshape(1, D), a_dst1.reshape(1, D))
    as0, ad0, as1, ad1 = scal[0], scal[1], scal[2], scal[3]
    out0, out1, dn0, dn1, ss0, ss1, cnt0, cnt1, _ = _sc_call(
        h0, h1, as0, ad0, as1, ad1, src_r, dst_r)
    feat, scores = _tc2(out0, out1,
                        dn0.reshape(N, 1), dn1.reshape(N, 1),
                        ss0.reshape(N, 1), ss1.reshape(N, 1),
                        cnt0.reshape(N, 1), cnt1.reshape(N, 1),
                        b0.reshape(1, D), b1.reshape(1, D))
    return (feat, scores.reshape(N))
